# Initial kernel scaffold; baseline (speedup 1.0000x reference)
import jax
import jax.numpy as jnp
from jax.experimental import pallas as pl


def kernel(pointcloud, params):
    raise NotImplementedError("write your pallas kernel here")


# speedup vs baseline: 7.8627x; 7.8627x over previous
import functools

import numpy as np
import jax
import jax.numpy as jnp
from jax import lax
from jax.experimental import pallas as pl
from jax.experimental.pallas import tpu as pltpu
from jax.experimental.pallas import tpu_sc as plsc

_B, _N = 8, 4096
_SA_SPECS = [
    (1024, (0.05, 0.1), (16, 32)),
    (256, (0.1, 0.2), (16, 32)),
    (64, (0.2, 0.4), (32, 64)),
]


def _fold(p):
  w = p['W']
  b = p['b']
  if 'gamma' in p:
    s = p['gamma'] / jnp.sqrt(p['var'] + 1e-5)
    w = w * s[:, None]
    b = (b - p['mean']) * s + p['beta']
  return jnp.transpose(w), b


def _fps_body(xyz_ref, cx_ref, cy_ref, cz_ref, *, npoint, n):
  rows = 8
  cols = n // rows
  x0 = xyz_ref[0, 0]
  x1 = xyz_ref[0, 1]
  x2 = xyz_ref[0, 2]
  flat = (lax.broadcasted_iota(jnp.int32, (rows, cols), 0) * cols
          + lax.broadcasted_iota(jnp.int32, (rows, cols), 1))

  def step(t, carry):
    dists, far = carry
    sel = flat == far
    c0 = jnp.sum(jnp.where(sel, x0, 0.0))
    c1 = jnp.sum(jnp.where(sel, x1, 0.0))
    c2 = jnp.sum(jnp.where(sel, x2, 0.0))
    cx_ref[0, pl.ds(t, 1), :] = jnp.reshape(c0, (1, 1))
    cy_ref[0, pl.ds(t, 1), :] = jnp.reshape(c1, (1, 1))
    cz_ref[0, pl.ds(t, 1), :] = jnp.reshape(c2, (1, 1))
    e0 = (x0 - c0) * (x0 - c0)
    e1 = (x1 - c1) * (x1 - c1)
    e2 = (x2 - c2) * (x2 - c2)
    d = (e0 + e1) + e2
    dists = jnp.minimum(dists, d)
    m = jnp.max(dists)
    far = jnp.min(jnp.where(dists == m, flat, n))
    return dists, far

  lax.fori_loop(0, npoint, step,
                (jnp.full((rows, cols), 1e10, jnp.float32), jnp.int32(0)))


def _fps(xyz_t, npoint):
  b, _, n = xyz_t.shape
  xyz4 = jnp.reshape(xyz_t, (b, 3, 8, n // 8))
  outs = pl.pallas_call(
      functools.partial(_fps_body, npoint=npoint, n=n),
      grid=(b,),
      in_specs=[pl.BlockSpec((1, 3, 8, n // 8), lambda i: (i, 0, 0, 0))],
      out_specs=[pl.BlockSpec((1, npoint, 1), lambda i: (i, 0, 0))] * 3,
      out_shape=[jax.ShapeDtypeStruct((b, npoint, 1), jnp.float32)] * 3,
  )(xyz4)
  return jnp.concatenate(outs, axis=-1)


def _sel_body(xyz_ref, q_ref, o1_ref, o2_ref, *, n, sb, r2s, nss):
  b = pl.program_id(0)
  xj0 = xyz_ref[0, 0:1, :]
  xj1 = xyz_ref[0, 1:2, :]
  xj2 = xyz_ref[0, 2:3, :]
  q = q_ref[0]
  q0 = q[:, 0:1]
  q1 = q[:, 1:2]
  q2 = q[:, 2:3]
  e0 = (q0 - xj0) * (q0 - xj0)
  e1 = (q1 - xj1) * (q1 - xj1)
  e2 = (q2 - xj2) * (q2 - xj2)
  d2 = (e0 + e1) + e2
  jot = lax.broadcasted_iota(jnp.int32, (sb, n), 1)
  base = b * n

  for r2, ns, o_ref in zip(r2s, nss, (o1_ref, o2_ref)):
    cur = jnp.where(d2 <= r2, jot, n)
    kiota = lax.broadcasted_iota(jnp.int32, (sb, ns), 1)
    idxm = jnp.zeros((sb, ns), jnp.int32)
    for k in range(ns):
      jk = jnp.min(cur, axis=1, keepdims=True)
      idxm = jnp.where(kiota == k, jk, idxm)
      cur = jnp.where(cur == jk, n, cur)
    first = idxm[:, :1]
    idxm = jnp.where(idxm == n, first, idxm)
    o_ref[0, :, :] = idxm + base


def _ball_select(xyz_t, new_xyz, radii, nss, sb):
  b, _, n = xyz_t.shape
  s = new_xyz.shape[1]
  r2s = tuple(np.float32(r * r) for r in radii)
  return pl.pallas_call(
      functools.partial(_sel_body, n=n, sb=sb, r2s=r2s, nss=tuple(nss)),
      grid=(b, s // sb),
      in_specs=[
          pl.BlockSpec((1, 3, n), lambda i, j: (i, 0, 0)),
          pl.BlockSpec((1, sb, 3), lambda i, j: (i, j, 0)),
      ],
      out_specs=[pl.BlockSpec((1, sb, ns), lambda i, j: (i, j, 0))
                 for ns in nss],
      out_shape=[jax.ShapeDtypeStruct((b, s, ns), jnp.int32) for ns in nss],
  )(xyz_t, new_xyz)


_CH = 128


def _sc_gather(table, idx):
  m = idx.shape[0]
  d = table.shape[1]
  info = plsc.get_sparse_core_info()
  nw = info.num_cores * info.num_subcores
  m_per_w = m // nw
  chunks = m_per_w // _CH
  mesh = plsc.VectorSubcoreMesh(core_axis_name="c", subcore_axis_name="s")

  @functools.partial(
      pl.kernel,
      out_type=jax.ShapeDtypeStruct((m, d), jnp.float32),
      mesh=mesh,
      compiler_params=pltpu.CompilerParams(use_tc_tiling_on_sc=False),
      scratch_types=[
          pltpu.VMEM((_CH,), jnp.int32),
          pltpu.VMEM((_CH, d), jnp.float32),
          pltpu.SemaphoreType.DMA,
      ],
  )
  def gk(table_hbm, idx_hbm, out_hbm, idx_v, rows_v, sem):
    wid = lax.axis_index("s") * info.num_cores + lax.axis_index("c")
    base = wid * m_per_w

    def chunk(i, carry):
      off = base + i * _CH
      pltpu.sync_copy(idx_hbm.at[pl.ds(off, _CH)], idx_v)
      pltpu.async_copy(table_hbm.at[idx_v], rows_v, sem).wait()
      pltpu.sync_copy(rows_v, out_hbm.at[pl.ds(off, _CH)])
      return carry

    lax.fori_loop(0, chunks, chunk, 0)

  return gk(table, idx)


def _chain_body(x_ref, *refs, acts):
  h = x_ref[...]
  nl = len(acts)
  for i in range(nl):
    wt = refs[i][...]
    bb = refs[nl + i][...]
    h = jnp.dot(h, wt, preferred_element_type=jnp.float32) + bb
    if acts[i]:
      h = jnp.maximum(h, 0.0)
  refs[2 * nl][...] = h


def _dense_chain(x2d, layers, block_rows):
  m, cin = x2d.shape
  wts = [l[0] for l in layers]
  bs = [jnp.reshape(l[1], (1, -1)) for l in layers]
  acts = tuple(l[2] for l in layers)
  cout = wts[-1].shape[1]
  grid = (m // block_rows,)
  in_specs = [pl.BlockSpec((block_rows, cin), lambda i: (i, 0))]
  for w in wts:
    in_specs.append(pl.BlockSpec(w.shape, lambda i: (0, 0)))
  for b in bs:
    in_specs.append(pl.BlockSpec(b.shape, lambda i: (0, 0)))
  return pl.pallas_call(
      functools.partial(_chain_body, acts=acts),
      grid=grid,
      in_specs=in_specs,
      out_specs=pl.BlockSpec((block_rows, cout), lambda i: (i, 0)),
      out_shape=jax.ShapeDtypeStruct((m, cout), jnp.float32),
  )(x2d, *wts, *bs)


def _samlp_body(g_ref, q_ref, w1x_ref, w2_ref, b2_ref, w3_ref, b3_ref, o_ref,
                *, qb, ns):
  corr = jnp.dot(q_ref[...], w1x_ref[...], preferred_element_type=jnp.float32)
  c1 = corr.shape[1]
  h1 = jnp.maximum(g_ref[...] - corr[:, None, :], 0.0)
  h1f = jnp.reshape(h1, (qb * ns, c1))
  h2 = jnp.maximum(
      jnp.dot(h1f, w2_ref[...], preferred_element_type=jnp.float32)
      + b2_ref[...], 0.0)
  h3 = jnp.maximum(
      jnp.dot(h2, w3_ref[...], preferred_element_type=jnp.float32)
      + b3_ref[...], 0.0)
  c3 = h3.shape[1]
  o_ref[...] = jnp.max(jnp.reshape(h3, (qb, ns, c3)), axis=1)


def _sa_mlp(g3d, new_xyz2d, w1x, w2, b2, w3, b3, qb):
  q, ns, c1 = g3d.shape
  c3 = w3.shape[1]
  b2r = jnp.reshape(b2, (1, -1))
  b3r = jnp.reshape(b3, (1, -1))
  return pl.pallas_call(
      functools.partial(_samlp_body, qb=qb, ns=ns),
      grid=(q // qb,),
      in_specs=[
          pl.BlockSpec((qb, ns, c1), lambda i: (i, 0, 0)),
          pl.BlockSpec((qb, 3), lambda i: (i, 0)),
          pl.BlockSpec(w1x.shape, lambda i: (0, 0)),
          pl.BlockSpec(w2.shape, lambda i: (0, 0)),
          pl.BlockSpec(b2r.shape, lambda i: (0, 0)),
          pl.BlockSpec(w3.shape, lambda i: (0, 0)),
          pl.BlockSpec(b3r.shape, lambda i: (0, 0)),
      ],
      out_specs=pl.BlockSpec((qb, c3), lambda i: (i, 0)),
      out_shape=jax.ShapeDtypeStruct((q, c3), jnp.float32),
  )(g3d, new_xyz2d, w1x, w2, b2r, w3, b3r)


def _mid_body(x_ref, w1_ref, b1_ref, w2_ref, b2_ref, o_ref, *, b, s):
  cin = x_ref.shape[2]
  h = jnp.reshape(x_ref[...], (b * s, cin))
  h = jnp.maximum(
      jnp.dot(h, w1_ref[...], preferred_element_type=jnp.float32)
      + b1_ref[...], 0.0)
  h = jnp.maximum(
      jnp.dot(h, w2_ref[...], preferred_element_type=jnp.float32)
      + b2_ref[...], 0.0)
  o_ref[...] = jnp.max(jnp.reshape(h, (b, s, h.shape[1])), axis=1)


def _mid(x3d, w1, b1, w2, b2):
  b, s, cin = x3d.shape
  cout = w2.shape[1]
  b1r = jnp.reshape(b1, (1, -1))
  b2r = jnp.reshape(b2, (1, -1))
  return pl.pallas_call(
      functools.partial(_mid_body, b=b, s=s),
      in_specs=[pl.BlockSpec(x3d.shape, lambda: (0, 0, 0)),
                pl.BlockSpec(w1.shape, lambda: (0, 0)),
                pl.BlockSpec(b1r.shape, lambda: (0, 0)),
                pl.BlockSpec(w2.shape, lambda: (0, 0)),
                pl.BlockSpec(b2r.shape, lambda: (0, 0))],
      out_specs=pl.BlockSpec((b, cout), lambda: (0, 0)),
      out_shape=jax.ShapeDtypeStruct((b, cout), jnp.float32),
  )(x3d, w1, b1r, w2, b2r)


def _fp_body(x1_ref, x2_ref, f1_ref, f2_ref, wa_ref, wb_ref, b1_ref,
             w2_ref, b2_ref, o_ref, *, s1b, s2):
  q = x1_ref[0]
  q0 = q[:, 0:1]
  q1 = q[:, 1:2]
  q2 = q[:, 2:3]
  p0 = x2_ref[0, 0:1, :]
  p1 = x2_ref[0, 1:2, :]
  p2 = x2_ref[0, 2:3, :]
  e0 = (q0 - p0) * (q0 - p0)
  e1 = (q1 - p1) * (q1 - p1)
  e2 = (q2 - p2) * (q2 - p2)
  d2 = (e0 + e1) + e2
  jot = lax.broadcasted_iota(jnp.int32, (s1b, s2), 1)
  vals = []
  idxs = []
  for _ in range(3):
    m = jnp.min(d2, axis=1, keepdims=True)
    jk = jnp.min(jnp.where(d2 == m, jot, s2), axis=1, keepdims=True)
    vals.append(m)
    idxs.append(jk)
    d2 = jnp.where(jot == jk, 1e10, d2)
  r0 = 1.0 / (vals[0] + 1e-8)
  r1 = 1.0 / (vals[1] + 1e-8)
  r2 = 1.0 / (vals[2] + 1e-8)
  den = (r0 + r1) + r2
  wm = jnp.where(jot == idxs[0], r0 / den, 0.0)
  wm = wm + jnp.where(jot == idxs[1], r1 / den, 0.0)
  wm = wm + jnp.where(jot == idxs[2], r2 / den, 0.0)
  interp = jnp.dot(wm, f2_ref[0], preferred_element_type=jnp.float32)
  h = (jnp.dot(interp, wa_ref[...], preferred_element_type=jnp.float32)
       + jnp.dot(f1_ref[0], wb_ref[...], preferred_element_type=jnp.float32)
       + b1_ref[...])
  h = jnp.maximum(h, 0.0)
  h = jnp.maximum(
      jnp.dot(h, w2_ref[...], preferred_element_type=jnp.float32)
      + b2_ref[...], 0.0)
  o_ref[0] = h


def _fp(xyz1, xyz2, feat1, feat2, layers, s1b):
  b, s1, _ = xyz1.shape
  s2 = xyz2.shape[1]
  xyz2_t = jnp.transpose(xyz2, (0, 2, 1))
  c2 = feat2.shape[2]
  c1f = feat1.shape[2]
  wt1, bb1 = layers[0]
  wa = wt1[:c2]
  wb = wt1[c2:]
  wt2, bb2 = layers[1]
  b1r = jnp.reshape(bb1, (1, -1))
  b2r = jnp.reshape(bb2, (1, -1))
  cout = wt2.shape[1]
  return pl.pallas_call(
      functools.partial(_fp_body, s1b=s1b, s2=s2),
      grid=(b, s1 // s1b),
      in_specs=[
          pl.BlockSpec((1, s1b, 3), lambda i, j: (i, j, 0)),
          pl.BlockSpec((1, 3, s2), lambda i, j: (i, 0, 0)),
          pl.BlockSpec((1, s1b, c1f), lambda i, j: (i, j, 0)),
          pl.BlockSpec((1, s2, c2), lambda i, j: (i, 0, 0)),
          pl.BlockSpec(wa.shape, lambda i, j: (0, 0)),
          pl.BlockSpec(wb.shape, lambda i, j: (0, 0)),
          pl.BlockSpec(b1r.shape, lambda i, j: (0, 0)),
          pl.BlockSpec(wt2.shape, lambda i, j: (0, 0)),
          pl.BlockSpec(b2r.shape, lambda i, j: (0, 0)),
      ],
      out_specs=pl.BlockSpec((1, s1b, cout), lambda i, j: (i, j, 0)),
      out_shape=jax.ShapeDtypeStruct((b, s1, cout), jnp.float32),
  )(xyz1, xyz2_t, feat1, feat2, wa, wb, b1r, wt2, b2r)


def _sa_level(xyz, feats, npoint, radii, nss, mlps_params, sb, qb):
  b, n, _ = xyz.shape
  xyz_t = jnp.transpose(xyz, (0, 2, 1))
  new_xyz = _fps(xyz_t, npoint)
  idx1, idx2 = _ball_select(xyz_t, new_xyz, radii, nss, sb)
  x2d = jnp.reshape(jnp.concatenate([xyz, feats], axis=-1), (b * n, -1))
  nq2d = jnp.reshape(new_xyz, (b * npoint, 3))
  outs = []
  for idx, ns, layers in zip((idx1, idx2), nss, mlps_params):
    (wt1, bb1), (wt2, bb2), (wt3, bb3) = layers
    p = _dense_chain(x2d, [(wt1, bb1, False)], block_rows=min(b * n, 2048))
    g = _sc_gather(p, jnp.reshape(idx, (-1,)))
    g3d = jnp.reshape(g, (b * npoint, ns, -1))
    o = _sa_mlp(g3d, nq2d, wt1[:3], wt2, bb2, wt3, bb3, qb)
    outs.append(jnp.reshape(o, (b, npoint, -1)))
  return new_xyz, jnp.concatenate(outs, axis=-1)


def kernel(pointcloud, params):
  xyz = pointcloud[..., :3]
  feats = pointcloud[..., 3:]
  fold = lambda ls: [_fold(p) for p in ls]

  l_xyz = [xyz]
  l_feat = [feats]
  sel_sb = [256, 256, 64]
  mlp_qb = [512, 256, 64]
  for i, (npo, radii, nss) in enumerate(_SA_SPECS):
    mlps = [fold(m) for m in params['sa%d' % (i + 1)]]
    nx, nf = _sa_level(l_xyz[i], l_feat[i], npo, radii, nss, mlps,
                       sel_sb[i], mlp_qb[i])
    l_xyz.append(nx)
    l_feat.append(nf)

  midp = fold(params['mid'])
  midx = jnp.concatenate([l_xyz[3], l_feat[3]], axis=-1)
  middle = _mid(midx, midp[0][0], midp[0][1], midp[1][0], midp[1][1])
  middle_features = jnp.reshape(middle, (middle.shape[0], 1, -1))

  fp_s1b = [1024, 1024, 256]
  names = ['fp0', 'fp1', 'fp2']
  for i in range(-1, -4, -1):
    layers = fold(params[names[i + 3]])
    l_feat[i - 1] = _fp(l_xyz[i - 1], l_xyz[i], l_feat[i - 1], l_feat[i],
                        layers, fp_s1b[i + 3])

  wf1, bf1 = _fold(params['fc1'])
  wf2, bf2 = _fold(params['fc2'])
  x2d = jnp.reshape(l_feat[0], (-1, wf1.shape[0]))
  logits = _dense_chain(x2d, [(wf1, bf1, True), (wf2, bf2, False)],
                        block_rows=2048)
  sem_logits = jnp.reshape(logits, (_B, _N, -1))
  return middle_features, sem_logits
